# baseline (speedup 1.0000x reference)
import functools

import jax
import jax.numpy as jnp
from jax import lax
from jax.experimental import pallas as pl
from jax.experimental.pallas import tpu as pltpu
from jax.experimental.pallas import tpu_sc as plsc

N = 10000
D = 128
E = 320000
H = D // 2
NC = 2
NS = 16
LANES = 16
EPT = E // NS
CH = 80
NR = 2
NCHUNK = EPT // (CH * NR)
NB = 5


def _sweep_body(x2, srcr, dstr, attrr, out, srcb, dstb, acc, gsem, ssem, *msgs):
    c = lax.axis_index("c")
    s = lax.axis_index("s")
    zero = jnp.zeros((LANES,), jnp.float32)

    def zfill(i, carry):
        for k in range(H // LANES):
            msgs[0][i, pl.ds(k * LANES, LANES)] = zero
        return carry

    lax.fori_loop(0, CH, zfill, 0)

    @pl.when(s < 10)
    def _zero_acc():
        for t in range(2000 // CH):
            pltpu.async_copy(msgs[0], acc.at[pl.ds(s * 2000 + t * CH, CH)], gsem.at[0])
        for t in range(2000 // CH):
            pltpu.make_async_copy(msgs[0], acc.at[pl.ds(0, CH)], gsem.at[0]).wait()

    plsc.subcore_barrier()

    def gwait(b):
        pltpu.make_async_copy(x2.at[srcb.at[0]], msgs[b], gsem.at[b]).wait()

    def swait(b):
        pltpu.make_async_copy(msgs[b], acc.at[dstb.at[0]], ssem.at[b]).wait()

    for r in range(NR):
        pltpu.sync_copy(attrr.at[s, r], dstb)
        pltpu.sync_copy(dstr.at[s, r], srcb)

        def sxbody(j, carry):
            for k in range(CH // LANES):
                sl = pl.ds(k * LANES, LANES)
                dstb[j, sl] = srcb[j, sl] * 2 + dstb[j, sl]
            return carry

        lax.fori_loop(0, NCHUNK, sxbody, 0)
        pltpu.sync_copy(srcr.at[s, r], srcb)

        def gxbody(j, carry):
            for k in range(CH // LANES):
                sl = pl.ds(k * LANES, LANES)
                srcb[j, sl] = srcb[j, sl] * 2 + c
            return carry

        lax.fori_loop(0, NCHUNK, gxbody, 0)

        for b in range(2):
            pltpu.async_copy(x2.at[srcb.at[b]], msgs[b], gsem.at[b])

        def gbody(g, carry):
            for b in range(NB):
                j = g * NB + b
                b2 = (b + 2) % NB

                @pl.when(j + 2 < NCHUNK)
                def _prefetch():
                    @pl.when(j - 3 >= 0)
                    def _drain():
                        swait(b2)

                    pltpu.async_copy(x2.at[srcb.at[j + 2]], msgs[b2], gsem.at[b2])

                gwait(b)
                pltpu.async_copy(msgs[b], acc.at[dstb.at[j]], ssem.at[b], add=True)
            return carry

        lax.fori_loop(0, NCHUNK // NB, gbody, 0)
        for b in range(NB):
            swait(b)

    plsc.subcore_barrier()

    @pl.when(s < 10)
    def _copy_out():
        pltpu.sync_copy(acc.at[pl.ds(s * 2000, 2000)], out.at[c, pl.ds(s * 2000, 2000)])


@functools.cache
def _sc_sweep():
    mesh = plsc.VectorSubcoreMesh(
        core_axis_name="c", subcore_axis_name="s", num_cores=NC, num_subcores=NS
    )
    return pl.kernel(
        _sweep_body,
        out_type=jax.ShapeDtypeStruct((NC, 2 * N, H), jnp.float32),
        mesh=mesh,
        compiler_params=pltpu.CompilerParams(use_tc_tiling_on_sc=False),
        scratch_types=[
            pltpu.VMEM((NCHUNK, CH), jnp.int32),
            pltpu.VMEM((NCHUNK, CH), jnp.int32),
            pltpu.VMEM_SHARED((2 * N, H), jnp.float32),
            pltpu.SemaphoreType.DMA((NB,)),
            pltpu.SemaphoreType.DMA((NB,)),
        ]
        + [pltpu.VMEM((CH, H), jnp.float32) for _ in range(NB)],
    )


RB = 1000
GRID = N // RB


def _dense0_body(x_ref, a0_ref, a1_ref, ws, bs, wk, bk, out_ref):
    xb = x_ref[...]
    agg = jnp.concatenate([a0_ref[0][:, :H], a1_ref[0][:, :H]], axis=1)
    t0 = jnp.maximum(jnp.dot(xb, ws[...], preferred_element_type=jnp.float32) + bs[...], 0.0)
    t1 = jnp.maximum(jnp.dot(agg, wk[...], preferred_element_type=jnp.float32) + bk[...], 0.0)
    out_ref[...] = xb + jnp.maximum(t0 + t1, 0.0)


def _dense0(x, xagg, Ws0, bs0, Wk0, bk0):
    wspec = pl.BlockSpec((D, D), lambda i: (0, 0))
    bspec = pl.BlockSpec((1, D), lambda i: (0, 0))
    return pl.pallas_call(
        _dense0_body,
        grid=(GRID,),
        in_specs=[
            pl.BlockSpec((RB, D), lambda i: (i, 0)),
            pl.BlockSpec((1, RB, D), lambda i: (0, i, 0)),
            pl.BlockSpec((1, RB, D), lambda i: (1, i, 0)),
            wspec, bspec, wspec, bspec,
        ],
        out_specs=pl.BlockSpec((RB, D), lambda i: (i, 0)),
        out_shape=jax.ShapeDtypeStruct((N, D), jnp.float32),
    )(x, xagg, xagg, Ws0, bs0, Wk0, bk0)


def _dense1_body(h_ref, g0, g1, x0, x1, ws, bs, wk1, bk1, wk2, bk2, out_ref):
    hb = h_ref[...]
    agg1 = jnp.concatenate([g0[0][:, :H], g1[0][:, :H]], axis=1)
    agg2 = jnp.concatenate([x0[0][:, H:], x1[0][:, H:]], axis=1)
    t0 = jnp.maximum(jnp.dot(hb, ws[...], preferred_element_type=jnp.float32) + bs[...], 0.0)
    t1 = jnp.maximum(jnp.dot(agg1, wk1[...], preferred_element_type=jnp.float32) + bk1[...], 0.0)
    t2 = jnp.maximum(jnp.dot(agg2, wk2[...], preferred_element_type=jnp.float32) + bk2[...], 0.0)
    out_ref[...] = hb + jnp.maximum(t0 + t1 + t2, 0.0)


def _dense1(h1, hagg, xagg, Ws1, bs1, Wk1, bk1, Wk2, bk2):
    wspec = pl.BlockSpec((D, D), lambda i: (0, 0))
    bspec = pl.BlockSpec((1, D), lambda i: (0, 0))
    return pl.pallas_call(
        _dense1_body,
        grid=(GRID,),
        in_specs=[
            pl.BlockSpec((RB, D), lambda i: (i, 0)),
            pl.BlockSpec((1, RB, D), lambda i: (0, i, 0)),
            pl.BlockSpec((1, RB, D), lambda i: (1, i, 0)),
            pl.BlockSpec((1, RB, D), lambda i: (0, i, 0)),
            pl.BlockSpec((1, RB, D), lambda i: (1, i, 0)),
            wspec, bspec, wspec, bspec, wspec, bspec,
        ],
        out_specs=pl.BlockSpec((RB, D), lambda i: (i, 0)),
        out_shape=jax.ShapeDtypeStruct((N, D), jnp.float32),
    )(h1, hagg, hagg, xagg, xagg, Ws1, bs1, Wk1, bk1, Wk2, bk2)


def kernel(x, edge_index, edge_attr, Ws0, bs0, Wk0_1, bk0_1, Ws1, bs1, Wk1_1, bk1_1, Wk1_2, bk1_2):
    src = edge_index[0].astype(jnp.int32).reshape(NS, NR, NCHUNK, CH)
    dst = edge_index[1].astype(jnp.int32).reshape(NS, NR, NCHUNK, CH)
    attr = edge_attr.astype(jnp.int32).reshape(NS, NR, NCHUNK, CH)
    sweep = _sc_sweep()
    xagg = sweep(x.reshape(2 * N, H), src, dst, attr).reshape(NC, N, D)
    h1 = _dense0(x, xagg, Ws0, bs0.reshape(1, D), Wk0_1, bk0_1.reshape(1, D))
    hagg = sweep(h1.reshape(2 * N, H), src, dst, attr).reshape(NC, N, D)
    return _dense1(
        h1, hagg, xagg,
        Ws1, bs1.reshape(1, D), Wk1_1, bk1_1.reshape(1, D), Wk1_2, bk1_2.reshape(1, D),
    )


# speedup vs baseline: 1.0189x; 1.0189x over previous
import functools

import jax
import jax.numpy as jnp
from jax import lax
from jax.experimental import pallas as pl
from jax.experimental.pallas import tpu as pltpu
from jax.experimental.pallas import tpu_sc as plsc

N = 10000
D = 128
E = 320000
H = D // 2
NC = 2
NS = 16
LANES = 16
EPT = E // NS
CH = 80
NR = 2
NCHUNK = EPT // (CH * NR)
NB = 5


def _sweep_body(x2, srcr, dstr, attrr, out, srcb, dstb, acc, gsem, ssem, *msgs):
    c = lax.axis_index("c")
    s = lax.axis_index("s")
    zero = jnp.zeros((LANES,), jnp.float32)

    def zfill(i, carry):
        for k in range(H // LANES):
            msgs[0][i, pl.ds(k * LANES, LANES)] = zero
        return carry

    lax.fori_loop(0, CH, zfill, 0)

    @pl.when(s < 10)
    def _zero_acc():
        for t in range(2000 // CH):
            pltpu.async_copy(msgs[0], acc.at[pl.ds(s * 2000 + t * CH, CH)], gsem.at[0])
        for t in range(2000 // CH):
            pltpu.make_async_copy(msgs[0], acc.at[pl.ds(0, CH)], gsem.at[0]).wait()

    plsc.subcore_barrier()

    def gwait(b):
        pltpu.make_async_copy(x2.at[srcb.at[0]], msgs[b], gsem.at[b]).wait()

    def swait(b):
        pltpu.make_async_copy(msgs[b], acc.at[dstb.at[0]], ssem.at[b]).wait()

    for r in range(NR):
        pltpu.sync_copy(attrr.at[s, r], dstb)
        pltpu.sync_copy(dstr.at[s, r], srcb)

        def sxbody(j, carry):
            for k in range(CH // LANES):
                sl = pl.ds(k * LANES, LANES)
                dstb[j, sl] = srcb[j, sl] * 2 + dstb[j, sl]
            return carry

        lax.fori_loop(0, NCHUNK, sxbody, 0)
        pltpu.sync_copy(srcr.at[s, r], srcb)

        def gxbody(j, carry):
            for k in range(CH // LANES):
                sl = pl.ds(k * LANES, LANES)
                srcb[j, sl] = srcb[j, sl] * 2 + c
            return carry

        lax.fori_loop(0, NCHUNK, gxbody, 0)

        for b in range(2):
            pltpu.async_copy(x2.at[srcb.at[b]], msgs[b], gsem.at[b])

        def gbody(g, carry):
            for b in range(NB):
                j = g * NB + b
                b2 = (b + 2) % NB

                @pl.when(j + 2 < NCHUNK)
                def _prefetch():
                    @pl.when(j - 3 >= 0)
                    def _drain():
                        swait(b2)

                    pltpu.async_copy(x2.at[srcb.at[j + 2]], msgs[b2], gsem.at[b2])

                gwait(b)
                pltpu.async_copy(msgs[b], acc.at[dstb.at[j]], ssem.at[b], add=True)
            return carry

        lax.fori_loop(0, NCHUNK // NB, gbody, 0)
        for b in range(NB):
            swait(b)

    plsc.subcore_barrier()

    @pl.when(s < 10)
    def _copy_out():
        pltpu.sync_copy(acc.at[pl.ds(s * 2000, 2000)], out.at[c, pl.ds(s * 2000, 2000)])


@functools.cache
def _sc_sweep():
    mesh = plsc.VectorSubcoreMesh(
        core_axis_name="c", subcore_axis_name="s", num_cores=NC, num_subcores=NS
    )
    return pl.kernel(
        _sweep_body,
        out_type=jax.ShapeDtypeStruct((NC, 2 * N, H), jnp.float32),
        mesh=mesh,
        compiler_params=pltpu.CompilerParams(use_tc_tiling_on_sc=False),
        scratch_types=[
            pltpu.VMEM((NCHUNK, CH), jnp.int32),
            pltpu.VMEM((NCHUNK, CH), jnp.int32),
            pltpu.VMEM_SHARED((2 * N, H), jnp.float32),
            pltpu.SemaphoreType.DMA((NB,)),
            pltpu.SemaphoreType.DMA((NB,)),
        ]
        + [pltpu.VMEM((CH, H), jnp.float32) for _ in range(NB)],
    )


RB = 2000
GRID = N // RB


def _dense0_body(x_ref, a0_ref, a1_ref, ws, bs, wk, bk, out_ref):
    xb = x_ref[...]
    agg = jnp.concatenate([a0_ref[0][:, :H], a1_ref[0][:, :H]], axis=1)
    t0 = jnp.maximum(jnp.dot(xb, ws[...], preferred_element_type=jnp.float32) + bs[...], 0.0)
    t1 = jnp.maximum(jnp.dot(agg, wk[...], preferred_element_type=jnp.float32) + bk[...], 0.0)
    out_ref[...] = xb + jnp.maximum(t0 + t1, 0.0)


def _dense0(x, xagg, Ws0, bs0, Wk0, bk0):
    wspec = pl.BlockSpec((D, D), lambda i: (0, 0))
    bspec = pl.BlockSpec((1, D), lambda i: (0, 0))
    return pl.pallas_call(
        _dense0_body,
        grid=(GRID,),
        in_specs=[
            pl.BlockSpec((RB, D), lambda i: (i, 0)),
            pl.BlockSpec((1, RB, D), lambda i: (0, i, 0)),
            pl.BlockSpec((1, RB, D), lambda i: (1, i, 0)),
            wspec, bspec, wspec, bspec,
        ],
        out_specs=pl.BlockSpec((RB, D), lambda i: (i, 0)),
        out_shape=jax.ShapeDtypeStruct((N, D), jnp.float32),
    )(x, xagg, xagg, Ws0, bs0, Wk0, bk0)


def _dense1_body(h_ref, g0, g1, x0, x1, ws, bs, wk1, bk1, wk2, bk2, out_ref):
    hb = h_ref[...]
    agg1 = jnp.concatenate([g0[0][:, :H], g1[0][:, :H]], axis=1)
    agg2 = jnp.concatenate([x0[0][:, H:], x1[0][:, H:]], axis=1)
    t0 = jnp.maximum(jnp.dot(hb, ws[...], preferred_element_type=jnp.float32) + bs[...], 0.0)
    t1 = jnp.maximum(jnp.dot(agg1, wk1[...], preferred_element_type=jnp.float32) + bk1[...], 0.0)
    t2 = jnp.maximum(jnp.dot(agg2, wk2[...], preferred_element_type=jnp.float32) + bk2[...], 0.0)
    out_ref[...] = hb + jnp.maximum(t0 + t1 + t2, 0.0)


def _dense1(h1, hagg, xagg, Ws1, bs1, Wk1, bk1, Wk2, bk2):
    wspec = pl.BlockSpec((D, D), lambda i: (0, 0))
    bspec = pl.BlockSpec((1, D), lambda i: (0, 0))
    return pl.pallas_call(
        _dense1_body,
        grid=(GRID,),
        in_specs=[
            pl.BlockSpec((RB, D), lambda i: (i, 0)),
            pl.BlockSpec((1, RB, D), lambda i: (0, i, 0)),
            pl.BlockSpec((1, RB, D), lambda i: (1, i, 0)),
            pl.BlockSpec((1, RB, D), lambda i: (0, i, 0)),
            pl.BlockSpec((1, RB, D), lambda i: (1, i, 0)),
            wspec, bspec, wspec, bspec, wspec, bspec,
        ],
        out_specs=pl.BlockSpec((RB, D), lambda i: (i, 0)),
        out_shape=jax.ShapeDtypeStruct((N, D), jnp.float32),
    )(h1, hagg, hagg, xagg, xagg, Ws1, bs1, Wk1, bk1, Wk2, bk2)


def kernel(x, edge_index, edge_attr, Ws0, bs0, Wk0_1, bk0_1, Ws1, bs1, Wk1_1, bk1_1, Wk1_2, bk1_2):
    src = edge_index[0].astype(jnp.int32).reshape(NS, NR, NCHUNK, CH)
    dst = edge_index[1].astype(jnp.int32).reshape(NS, NR, NCHUNK, CH)
    attr = edge_attr.astype(jnp.int32).reshape(NS, NR, NCHUNK, CH)
    sweep = _sc_sweep()
    xagg = sweep(x.reshape(2 * N, H), src, dst, attr).reshape(NC, N, D)
    h1 = _dense0(x, xagg, Ws0, bs0.reshape(1, D), Wk0_1, bk0_1.reshape(1, D))
    hagg = sweep(h1.reshape(2 * N, H), src, dst, attr).reshape(NC, N, D)
    return _dense1(
        h1, hagg, xagg,
        Ws1, bs1.reshape(1, D), Wk1_1, bk1_1.reshape(1, D), Wk1_2, bk1_2.reshape(1, D),
    )


# speedup vs baseline: 1.0248x; 1.0058x over previous
import functools

import jax
import jax.numpy as jnp
from jax import lax
from jax.experimental import pallas as pl
from jax.experimental.pallas import tpu as pltpu
from jax.experimental.pallas import tpu_sc as plsc

N = 10000
D = 128
E = 320000
H = D // 2
NC = 2
NS = 16
LANES = 16
EPT = E // NS
CH = 80
NR = 2
NCHUNK = EPT // (CH * NR)
NB = 5


def _sweep_body(x2, srcr, dstr, attrr, out, srcb, dstb, acc, gsem, ssem, *msgs):
    c = lax.axis_index("c")
    s = lax.axis_index("s")
    zero = jnp.zeros((LANES,), jnp.float32)

    def zfill(i, carry):
        for k in range(H // LANES):
            msgs[0][i, pl.ds(k * LANES, LANES)] = zero
        return carry

    lax.fori_loop(0, CH, zfill, 0)

    @pl.when(s < 10)
    def _zero_acc():
        for t in range(2000 // CH):
            pltpu.async_copy(msgs[0], acc.at[pl.ds(s * 2000 + t * CH, CH)], gsem.at[0])
        for t in range(2000 // CH):
            pltpu.make_async_copy(msgs[0], acc.at[pl.ds(0, CH)], gsem.at[0]).wait()

    plsc.subcore_barrier()

    def gwait(b):
        pltpu.make_async_copy(x2.at[srcb.at[0]], msgs[b], gsem.at[b]).wait()

    def swait(b):
        pltpu.make_async_copy(msgs[b], acc.at[dstb.at[0]], ssem.at[b]).wait()

    for r in range(NR):
        pltpu.sync_copy(attrr.at[s, r], dstb)
        pltpu.sync_copy(dstr.at[s, r], srcb)

        def sxbody(j, carry):
            for k in range(CH // LANES):
                sl = pl.ds(k * LANES, LANES)
                dstb[j, sl] = srcb[j, sl] * 2 + dstb[j, sl]
            return carry

        lax.fori_loop(0, NCHUNK, sxbody, 0)
        pltpu.sync_copy(srcr.at[s, r], srcb)

        def gxbody(j, carry):
            for k in range(CH // LANES):
                sl = pl.ds(k * LANES, LANES)
                srcb[j, sl] = srcb[j, sl] * 2 + c
            return carry

        lax.fori_loop(0, NCHUNK, gxbody, 0)

        for b in range(2):
            pltpu.async_copy(x2.at[srcb.at[b]], msgs[b], gsem.at[b])

        def gbody(g, carry):
            for b in range(NB):
                j = g * NB + b
                b2 = (b + 2) % NB

                @pl.when(j + 2 < NCHUNK)
                def _prefetch():
                    @pl.when(j - 3 >= 0)
                    def _drain():
                        swait(b2)

                    pltpu.async_copy(x2.at[srcb.at[j + 2]], msgs[b2], gsem.at[b2])

                gwait(b)
                pltpu.async_copy(msgs[b], acc.at[dstb.at[j]], ssem.at[b], add=True)
            return carry

        lax.fori_loop(0, NCHUNK // NB, gbody, 0)
        for b in range(NB):
            swait(b)

    plsc.subcore_barrier()

    @pl.when(s < 10)
    def _copy_out():
        pltpu.sync_copy(acc.at[pl.ds(s * 2000, 2000)], out.at[c, pl.ds(s * 2000, 2000)])


@functools.cache
def _sc_sweep():
    mesh = plsc.VectorSubcoreMesh(
        core_axis_name="c", subcore_axis_name="s", num_cores=NC, num_subcores=NS
    )
    return pl.kernel(
        _sweep_body,
        out_type=jax.ShapeDtypeStruct((NC, 2 * N, H), jnp.float32),
        mesh=mesh,
        compiler_params=pltpu.CompilerParams(use_tc_tiling_on_sc=False),
        scratch_types=[
            pltpu.VMEM((NCHUNK, CH), jnp.int32),
            pltpu.VMEM((NCHUNK, CH), jnp.int32),
            pltpu.VMEM_SHARED((2 * N, H), jnp.float32),
            pltpu.SemaphoreType.DMA((NB,)),
            pltpu.SemaphoreType.DMA((NB,)),
        ]
        + [pltpu.VMEM((CH, H), jnp.float32) for _ in range(NB)],
    )


RB = 5000
GRID = N // RB


def _dense0_body(x_ref, a0_ref, a1_ref, ws, bs, wk, bk, out_ref):
    xb = x_ref[...]
    agg = jnp.concatenate([a0_ref[0][:, :H], a1_ref[0][:, :H]], axis=1)
    t0 = jnp.maximum(jnp.dot(xb, ws[...], preferred_element_type=jnp.float32) + bs[...], 0.0)
    t1 = jnp.maximum(jnp.dot(agg, wk[...], preferred_element_type=jnp.float32) + bk[...], 0.0)
    out_ref[...] = xb + jnp.maximum(t0 + t1, 0.0)


def _dense0(x, xagg, Ws0, bs0, Wk0, bk0):
    wspec = pl.BlockSpec((D, D), lambda i: (0, 0))
    bspec = pl.BlockSpec((1, D), lambda i: (0, 0))
    return pl.pallas_call(
        _dense0_body,
        grid=(GRID,),
        in_specs=[
            pl.BlockSpec((RB, D), lambda i: (i, 0)),
            pl.BlockSpec((1, RB, D), lambda i: (0, i, 0)),
            pl.BlockSpec((1, RB, D), lambda i: (1, i, 0)),
            wspec, bspec, wspec, bspec,
        ],
        out_specs=pl.BlockSpec((RB, D), lambda i: (i, 0)),
        out_shape=jax.ShapeDtypeStruct((N, D), jnp.float32),
    )(x, xagg, xagg, Ws0, bs0, Wk0, bk0)


def _dense1_body(h_ref, g0, g1, x0, x1, ws, bs, wk1, bk1, wk2, bk2, out_ref):
    hb = h_ref[...]
    agg1 = jnp.concatenate([g0[0][:, :H], g1[0][:, :H]], axis=1)
    agg2 = jnp.concatenate([x0[0][:, H:], x1[0][:, H:]], axis=1)
    t0 = jnp.maximum(jnp.dot(hb, ws[...], preferred_element_type=jnp.float32) + bs[...], 0.0)
    t1 = jnp.maximum(jnp.dot(agg1, wk1[...], preferred_element_type=jnp.float32) + bk1[...], 0.0)
    t2 = jnp.maximum(jnp.dot(agg2, wk2[...], preferred_element_type=jnp.float32) + bk2[...], 0.0)
    out_ref[...] = hb + jnp.maximum(t0 + t1 + t2, 0.0)


def _dense1(h1, hagg, xagg, Ws1, bs1, Wk1, bk1, Wk2, bk2):
    wspec = pl.BlockSpec((D, D), lambda i: (0, 0))
    bspec = pl.BlockSpec((1, D), lambda i: (0, 0))
    return pl.pallas_call(
        _dense1_body,
        grid=(GRID,),
        in_specs=[
            pl.BlockSpec((RB, D), lambda i: (i, 0)),
            pl.BlockSpec((1, RB, D), lambda i: (0, i, 0)),
            pl.BlockSpec((1, RB, D), lambda i: (1, i, 0)),
            pl.BlockSpec((1, RB, D), lambda i: (0, i, 0)),
            pl.BlockSpec((1, RB, D), lambda i: (1, i, 0)),
            wspec, bspec, wspec, bspec, wspec, bspec,
        ],
        out_specs=pl.BlockSpec((RB, D), lambda i: (i, 0)),
        out_shape=jax.ShapeDtypeStruct((N, D), jnp.float32),
    )(h1, hagg, hagg, xagg, xagg, Ws1, bs1, Wk1, bk1, Wk2, bk2)


def kernel(x, edge_index, edge_attr, Ws0, bs0, Wk0_1, bk0_1, Ws1, bs1, Wk1_1, bk1_1, Wk1_2, bk1_2):
    src = edge_index[0].astype(jnp.int32).reshape(NS, NR, NCHUNK, CH)
    dst = edge_index[1].astype(jnp.int32).reshape(NS, NR, NCHUNK, CH)
    attr = edge_attr.astype(jnp.int32).reshape(NS, NR, NCHUNK, CH)
    sweep = _sc_sweep()
    xagg = sweep(x.reshape(2 * N, H), src, dst, attr).reshape(NC, N, D)
    h1 = _dense0(x, xagg, Ws0, bs0.reshape(1, D), Wk0_1, bk0_1.reshape(1, D))
    hagg = sweep(h1.reshape(2 * N, H), src, dst, attr).reshape(NC, N, D)
    return _dense1(
        h1, hagg, xagg,
        Ws1, bs1.reshape(1, D), Wk1_1, bk1_1.reshape(1, D), Wk1_2, bk1_2.reshape(1, D),
    )
